# baseline (speedup 1.0000x reference)
import jax
import jax.numpy as jnp
from jax import lax
from jax.experimental import pallas as pl
from jax.experimental.pallas import tpu as pltpu

_C = 768
_K = 77
_L = 512
_INT_MIN = -2147483648
_INT_MAX = 2147483647


def _kw_block(dc_ref, x_ref, o_ref, u31_ref):
    xb = x_ref[0]
    scale = jnp.exp(-dc_ref[...])
    boosted = xb * scale

    s = lax.bitcast_convert_type(boosted, jnp.int32)
    skey = jnp.where(s < 0, s ^ jnp.int32(0x7FFFFFFF), s)
    u31_ref[...] = lax.shift_right_logical(skey ^ jnp.int32(_INT_MIN), 1)

    L = xb.shape[1]

    def count31_ge(cand):
        u31 = u31_ref[...]
        return jnp.sum((u31 >= cand).astype(jnp.int32), axis=0, keepdims=True)

    lo0 = jnp.zeros((1, L), jnp.int32)
    hi0 = jnp.full((1, L), jnp.int32(_INT_MAX))
    clo0 = jnp.full((1, L), jnp.int32(_C))
    chi0 = jnp.zeros((1, L), jnp.int32)
    r0 = jnp.zeros((1, L), jnp.int32)
    hasr0 = jnp.zeros((1, L), jnp.int32)
    coll0 = jnp.zeros((1, L), jnp.int32)

    def cond(state):
        it, lo, hi, clo, chi, r, hasr, coll = state
        return jnp.logical_and(it < 70, jnp.min(hasr | coll) == 0)

    def body(state):
        it, lo, hi, clo, chi, r, hasr, coll = state
        done = (hasr | coll) > 0

        mid = lo + lax.shift_right_logical(hi - lo, 1)
        flo = lo.astype(jnp.float32)
        fhi = hi.astype(jnp.float32)
        frac = (clo - _K).astype(jnp.float32) / (
            (clo - chi).astype(jnp.float32))
        interp = (flo + (fhi - flo) * frac).astype(jnp.int32)
        cand = jnp.where(it % 2 == 0, mid, interp)
        cand = jnp.clip(cand, lo + 1, hi - 1)
        cand = jnp.where(done, lo, cand)

        cnt = count31_ge(cand)

        hit = jnp.logical_and(cnt == _K, jnp.logical_not(done))
        r = jnp.where(hit, cand, r)
        hasr = hasr | hit.astype(jnp.int32)

        ge = cnt >= _K
        upd = jnp.logical_not(jnp.logical_or(done, hit))
        lo = jnp.where(jnp.logical_and(upd, ge), cand, lo)
        clo = jnp.where(jnp.logical_and(upd, ge), cnt, clo)
        hi = jnp.where(jnp.logical_and(upd, jnp.logical_not(ge)), cand, hi)
        chi = jnp.where(jnp.logical_and(upd, jnp.logical_not(ge)), cnt, chi)
        coll = coll | jnp.logical_and(hasr == 0,
                                      hi - lo <= 1).astype(jnp.int32)
        return it + 1, lo, hi, clo, chi, r, hasr, coll

    _, lo, hi, clo, chi, r, hasr, coll = lax.while_loop(
        cond, body, (jnp.int32(0), lo0, hi0, clo0, chi0, r0, hasr0, coll0))

    r_skey = (r << 1) ^ jnp.int32(_INT_MIN)
    mask_r = skey >= r_skey
    t_min = jnp.min(jnp.where(mask_r, skey, jnp.int32(_INT_MAX)),
                    axis=0, keepdims=True)
    c_skey = ((lo << 1) | 1) ^ jnp.int32(_INT_MIN)
    cntc = jnp.sum((skey >= c_skey).astype(jnp.int32), axis=0, keepdims=True)
    t_coll = jnp.where(cntc >= _K, c_skey, c_skey - 1)

    t_skey = jnp.where(hasr > 0, t_min, t_coll)
    s_t = jnp.where(t_skey < 0, t_skey ^ jnp.int32(0x7FFFFFFF), t_skey)
    thresh = lax.bitcast_convert_type(s_t, jnp.float32)
    o_ref[0] = jnp.where(boosted < thresh, jnp.zeros_like(xb), xb)


def kernel(x, duty_cycles):
    B, C, H, W = x.shape
    hw = H * W
    x3 = x.reshape(B, C, hw)
    dc = duty_cycles.reshape(C, 1)
    out = pl.pallas_call(
        _kw_block,
        grid=(B, hw // _L),
        in_specs=[
            pl.BlockSpec((C, 1), lambda b, j: (0, 0)),
            pl.BlockSpec((1, C, _L), lambda b, j: (b, 0, j)),
        ],
        out_specs=pl.BlockSpec((1, C, _L), lambda b, j: (b, 0, j)),
        out_shape=jax.ShapeDtypeStruct((B, C, hw), jnp.float32),
        scratch_shapes=[pltpu.VMEM((C, _L), jnp.int32)],
        compiler_params=pltpu.CompilerParams(
            dimension_semantics=("parallel", "parallel"),
        ),
    )(dc, x3)
    return out.reshape(B, C, H, W)


# speedup vs baseline: 1.4781x; 1.4781x over previous
import jax
import jax.numpy as jnp
from jax import lax
from jax.experimental import pallas as pl
from jax.experimental.pallas import tpu as pltpu

_C = 768
_K = 77
_L = 512
_INT_MIN = -2147483648


def _kw_block(dc_ref, x_ref, o_ref):
    xb = x_ref[0]
    scale = jnp.exp(-dc_ref[...])
    boosted = xb * scale

    s = lax.bitcast_convert_type(boosted, jnp.int32)
    skey = jnp.where(s < 0, s ^ jnp.int32(0x7FFFFFFF), s)

    ones = jnp.ones((1, _C), jnp.float32)
    kf = jnp.float32(_K)
    halves = [skey[:, :_L // 2], skey[:, _L // 2:]]

    def count_ge(sk, cand):
        ind = (sk >= cand).astype(jnp.float32)
        return lax.dot_general(ones, ind, (((1,), (0,)), ((), ())),
                               preferred_element_type=jnp.float32)

    ps = []
    for sk in halves:
        zero = jnp.zeros((1, sk.shape[1]), jnp.int32)
        ps.append(jnp.where(count_ge(sk, zero) >= kf, zero,
                            jnp.full_like(zero, jnp.int32(_INT_MIN))))
    for bit in range(30, -1, -1):
        for h in range(2):
            cand = ps[h] | jnp.int32(1 << bit)
            ps[h] = jnp.where(count_ge(halves[h], cand) >= kf, cand, ps[h])

    p = jnp.concatenate(ps, axis=1)
    s_t = jnp.where(p < 0, p ^ jnp.int32(0x7FFFFFFF), p)
    thresh = lax.bitcast_convert_type(s_t, jnp.float32)
    o_ref[0] = jnp.where(boosted < thresh, jnp.zeros_like(xb), xb)


def kernel(x, duty_cycles):
    B, C, H, W = x.shape
    hw = H * W
    x3 = x.reshape(B, C, hw)
    dc = duty_cycles.reshape(C, 1)
    out = pl.pallas_call(
        _kw_block,
        grid=(B, hw // _L),
        in_specs=[
            pl.BlockSpec((C, 1), lambda b, j: (0, 0)),
            pl.BlockSpec((1, C, _L), lambda b, j: (b, 0, j)),
        ],
        out_specs=pl.BlockSpec((1, C, _L), lambda b, j: (b, 0, j)),
        out_shape=jax.ShapeDtypeStruct((B, C, hw), jnp.float32),
        compiler_params=pltpu.CompilerParams(
            dimension_semantics=("parallel", "parallel"),
        ),
    )(dc, x3)
    return out.reshape(B, C, H, W)


# speedup vs baseline: 1.7892x; 1.2105x over previous
import jax
import jax.numpy as jnp
from jax.experimental import pallas as pl
from jax.experimental.pallas import tpu as pltpu

_C = 768
_K = 77
_L = 512
_INT_MIN = -2147483648


def _kw_block(dc_ref, x_ref, o_ref):
    xb = x_ref[0]
    scale = jnp.exp(-dc_ref[...])
    boosted = xb * scale

    s = jax.lax.bitcast_convert_type(boosted, jnp.int32)
    skey = jnp.where(s < 0, s ^ jnp.int32(0x7FFFFFFF), s)

    def count_ge(cand):
        return jnp.sum((skey >= cand).astype(jnp.int32), axis=0, keepdims=True)

    zero = jnp.zeros((1, xb.shape[1]), jnp.int32)
    p = jnp.where(count_ge(zero) >= _K, zero, jnp.full_like(zero, jnp.int32(_INT_MIN)))
    for bit in range(30, -1, -1):
        cand = p | jnp.int32(1 << bit)
        p = jnp.where(count_ge(cand) >= _K, cand, p)

    s_t = jnp.where(p < 0, p ^ jnp.int32(0x7FFFFFFF), p)
    thresh = jax.lax.bitcast_convert_type(s_t, jnp.float32)
    o_ref[0] = jnp.where(boosted < thresh, jnp.zeros_like(xb), xb)


def kernel(x, duty_cycles):
    B, C, H, W = x.shape
    hw = H * W
    x3 = x.reshape(B, C, hw)
    dc = duty_cycles.reshape(C, 1)
    out = pl.pallas_call(
        _kw_block,
        grid=(B, hw // _L),
        in_specs=[
            pl.BlockSpec((C, 1), lambda b, j: (0, 0)),
            pl.BlockSpec((1, C, _L), lambda b, j: (b, 0, j)),
        ],
        out_specs=pl.BlockSpec((1, C, _L), lambda b, j: (b, 0, j)),
        out_shape=jax.ShapeDtypeStruct((B, C, hw), jnp.float32),
        compiler_params=pltpu.CompilerParams(
            dimension_semantics=("parallel", "parallel"),
        ),
    )(dc, x3)
    return out.reshape(B, C, H, W)
